# Initial kernel scaffold; baseline (speedup 1.0000x reference)
import jax
import jax.numpy as jnp
from jax.experimental import pallas as pl


def kernel(x, E):
    raise NotImplementedError("write your pallas kernel here")


# speedup vs baseline: 1.3964x; 1.3964x over previous
import jax
import jax.numpy as jnp
from jax.experimental import pallas as pl

BLOCK_ROWS = 256


def _add_kernel(x_ref, e_ref, o_ref):
    o_ref[...] = x_ref[...] + e_ref[...]


def kernel(x, E):
    B, S, D = x.shape
    grid = (B, S // BLOCK_ROWS)
    return pl.pallas_call(
        _add_kernel,
        grid=grid,
        in_specs=[
            pl.BlockSpec((1, BLOCK_ROWS, D), lambda b, i: (b, i, 0)),
            pl.BlockSpec((BLOCK_ROWS, D), lambda b, i: (i, 0)),
        ],
        out_specs=pl.BlockSpec((1, BLOCK_ROWS, D), lambda b, i: (b, i, 0)),
        out_shape=jax.ShapeDtypeStruct(x.shape, x.dtype),
    )(x, E)


# speedup vs baseline: 1.4859x; 1.0641x over previous
import jax
import jax.numpy as jnp
from jax.experimental import pallas as pl

BLOCK_ROWS = 256


def _add_kernel(x_ref, e_ref, o_ref):
    o_ref[...] = x_ref[...] + e_ref[...]


def kernel(x, E):
    B, S, D = x.shape
    grid = (S // BLOCK_ROWS, B)
    return pl.pallas_call(
        _add_kernel,
        grid=grid,
        in_specs=[
            pl.BlockSpec((1, BLOCK_ROWS, D), lambda i, b: (b, i, 0)),
            pl.BlockSpec((BLOCK_ROWS, D), lambda i, b: (i, 0)),
        ],
        out_specs=pl.BlockSpec((1, BLOCK_ROWS, D), lambda i, b: (b, i, 0)),
        out_shape=jax.ShapeDtypeStruct(x.shape, x.dtype),
    )(x, E)


# speedup vs baseline: 1.9287x; 1.2980x over previous
import jax
import jax.numpy as jnp
from jax.experimental import pallas as pl

BLOCK_ROWS = 512


def _add_kernel(x_ref, e_ref, o_ref):
    o_ref[...] = x_ref[...] + e_ref[...]


def kernel(x, E):
    B, S, D = x.shape
    grid = (S // BLOCK_ROWS, B)
    return pl.pallas_call(
        _add_kernel,
        grid=grid,
        in_specs=[
            pl.BlockSpec((1, BLOCK_ROWS, D), lambda i, b: (b, i, 0)),
            pl.BlockSpec((BLOCK_ROWS, D), lambda i, b: (i, 0)),
        ],
        out_specs=pl.BlockSpec((1, BLOCK_ROWS, D), lambda i, b: (b, i, 0)),
        out_shape=jax.ShapeDtypeStruct(x.shape, x.dtype),
    )(x, E)


# speedup vs baseline: 2.1060x; 1.0919x over previous
import jax
import jax.numpy as jnp
from jax.experimental import pallas as pl

BLOCK_ROWS = 1024


def _add_kernel(x_ref, e_ref, o_ref):
    o_ref[...] = x_ref[...] + e_ref[...]


def kernel(x, E):
    B, S, D = x.shape
    grid = (S // BLOCK_ROWS, B)
    return pl.pallas_call(
        _add_kernel,
        grid=grid,
        in_specs=[
            pl.BlockSpec((1, BLOCK_ROWS, D), lambda i, b: (b, i, 0)),
            pl.BlockSpec((BLOCK_ROWS, D), lambda i, b: (i, 0)),
        ],
        out_specs=pl.BlockSpec((1, BLOCK_ROWS, D), lambda i, b: (b, i, 0)),
        out_shape=jax.ShapeDtypeStruct(x.shape, x.dtype),
    )(x, E)


# speedup vs baseline: 2.2625x; 1.0743x over previous
import jax
import jax.numpy as jnp
from jax.experimental import pallas as pl

BLOCK_ROWS = 2048


def _add_kernel(x_ref, e_ref, o_ref):
    o_ref[...] = x_ref[...] + e_ref[...]


def kernel(x, E):
    B, S, D = x.shape
    grid = (S // BLOCK_ROWS, B)
    return pl.pallas_call(
        _add_kernel,
        grid=grid,
        in_specs=[
            pl.BlockSpec((1, BLOCK_ROWS, D), lambda i, b: (b, i, 0)),
            pl.BlockSpec((BLOCK_ROWS, D), lambda i, b: (i, 0)),
        ],
        out_specs=pl.BlockSpec((1, BLOCK_ROWS, D), lambda i, b: (b, i, 0)),
        out_shape=jax.ShapeDtypeStruct(x.shape, x.dtype),
    )(x, E)
